# baseline (speedup 1.0000x reference)
import jax
import jax.numpy as jnp
from jax import lax
from jax.experimental import pallas as pl
from jax.experimental.pallas import tpu as pltpu

N = 2048
H = 128
E = 4
I = 512
NT = N // H
Q = N * E // H


def _body(h_blk_ref, adj_ref, edge_ref, h_ref, A_ref, WeU_ref, wbU_ref,
          UwT_ref, Ub_ref, out_ref, aggh_ref, deg_ref, r3_ref):
    k = pl.program_id(0)

    @pl.when(k == 0)
    def _init():
        aggh_ref[...] = jnp.zeros_like(aggh_ref)
        deg_ref[...] = jnp.zeros_like(deg_ref)
        r3_ref[...] = jnp.zeros_like(r3_ref)

    mask = (adj_ref[...] == 1).astype(jnp.float32)

    aggh_ref[...] += lax.dot_general(
        mask, h_blk_ref[...], (((0,), (0,)), ((), ())),
        preferred_element_type=jnp.float32)

    ones_col = jnp.ones((I, 1), dtype=jnp.float32)
    deg_ref[...] += lax.dot_general(
        mask, ones_col, (((0,), (0,)), ((), ())),
        preferred_element_type=jnp.float32)

    maskE = mask.reshape(I, NT, H)
    mask3 = jnp.repeat(maskE, E, axis=1)
    r3_ref[...] += jnp.sum(mask3 * edge_ref[...], axis=0)

    @pl.when(k == pl.num_programs(0) - 1)
    def _epilogue():
        ii = lax.broadcasted_iota(jnp.int32, (H, H), 0)
        jj = lax.broadcasted_iota(jnp.int32, (H, H), 1)
        ident = (ii == jj).astype(jnp.float32)
        r3T = lax.dot_general(ident, r3_ref[...], (((1,), (1,)), ((), ())),
                              preferred_element_type=jnp.float32)

        msg = (lax.dot_general(aggh_ref[...], A_ref[...],
                               (((1,), (0,)), ((), ())),
                               preferred_element_type=jnp.float32)
               + lax.dot_general(deg_ref[...], wbU_ref[...],
                                 (((1,), (0,)), ((), ())),
                                 preferred_element_type=jnp.float32)
               + lax.dot_general(h_ref[...], UwT_ref[...],
                                 (((1,), (0,)), ((), ())),
                                 preferred_element_type=jnp.float32)
               + Ub_ref[...])

        for jt in range(NT):
            blk = lax.dot_general(r3T[:, E * jt:E * (jt + 1)], WeU_ref[...],
                                  (((1,), (0,)), ((), ())),
                                  preferred_element_type=jnp.float32)
            out_ref[jt * H:(jt + 1) * H, :] = msg[jt * H:(jt + 1) * H, :] + blk


@jax.jit
def kernel(h, edge_attr, adj, W_w, W_b, U_w, U_b):
    edge3 = (edge_attr.reshape(N, NT, H, E)
             .transpose(0, 1, 3, 2)
             .reshape(N, Q, H))

    UwT = U_w.T
    A = W_w[:, :H].T @ UwT
    WeU = W_w[:, H:].T @ UwT
    wbU = (W_b @ UwT)[None, :]
    Ub = U_b[None, :]

    out = pl.pallas_call(
        _body,
        grid=(N // I,),
        in_specs=[
            pl.BlockSpec((I, H), lambda k: (k, 0)),
            pl.BlockSpec((I, N), lambda k: (k, 0)),
            pl.BlockSpec((I, Q, H), lambda k: (k, 0, 0)),
            pl.BlockSpec((N, H), lambda k: (0, 0)),
            pl.BlockSpec((H, H), lambda k: (0, 0)),
            pl.BlockSpec((E, H), lambda k: (0, 0)),
            pl.BlockSpec((1, H), lambda k: (0, 0)),
            pl.BlockSpec((H, H), lambda k: (0, 0)),
            pl.BlockSpec((1, H), lambda k: (0, 0)),
        ],
        out_specs=pl.BlockSpec((N, H), lambda k: (0, 0)),
        out_shape=jax.ShapeDtypeStruct((N, H), jnp.float32),
        scratch_shapes=[
            pltpu.VMEM((N, H), jnp.float32),
            pltpu.VMEM((N, 1), jnp.float32),
            pltpu.VMEM((Q, H), jnp.float32),
        ],
    )(h, adj, edge3, h, A, WeU, wbU, UwT, Ub)
    return out


# speedup vs baseline: 1.0111x; 1.0111x over previous
import jax
import jax.numpy as jnp
from jax import lax
from jax.experimental import pallas as pl
from jax.experimental.pallas import tpu as pltpu

N = 2048
H = 128
E = 4
I = 128
NT = N // H
Q = N * E // H


def _body(h_blk_ref, adj_ref, edge_ref, h_ref, A_ref, WeU_ref, wbU_ref,
          UwT_ref, Ub_ref, out_ref, aggh_ref, deg_ref, r3_ref):
    k = pl.program_id(0)

    @pl.when(k == 0)
    def _init():
        aggh_ref[...] = jnp.zeros_like(aggh_ref)
        deg_ref[...] = jnp.zeros_like(deg_ref)
        r3_ref[...] = jnp.zeros_like(r3_ref)

    mask = (adj_ref[...] == 1).astype(jnp.float32)

    aggh_ref[...] += lax.dot_general(
        mask, h_blk_ref[...], (((0,), (0,)), ((), ())),
        preferred_element_type=jnp.float32)

    ones_col = jnp.ones((I, 1), dtype=jnp.float32)
    deg_ref[...] += lax.dot_general(
        mask, ones_col, (((0,), (0,)), ((), ())),
        preferred_element_type=jnp.float32)

    maskE = mask.reshape(I, NT, H)
    mask3 = jnp.repeat(maskE, E, axis=1)
    r3_ref[...] += jnp.sum(mask3 * edge_ref[...], axis=0)

    @pl.when(k == pl.num_programs(0) - 1)
    def _epilogue():
        ii = lax.broadcasted_iota(jnp.int32, (H, H), 0)
        jj = lax.broadcasted_iota(jnp.int32, (H, H), 1)
        ident = (ii == jj).astype(jnp.float32)
        r3T = lax.dot_general(ident, r3_ref[...], (((1,), (1,)), ((), ())),
                              preferred_element_type=jnp.float32)

        msg = (lax.dot_general(aggh_ref[...], A_ref[...],
                               (((1,), (0,)), ((), ())),
                               preferred_element_type=jnp.float32)
               + lax.dot_general(deg_ref[...], wbU_ref[...],
                                 (((1,), (0,)), ((), ())),
                                 preferred_element_type=jnp.float32)
               + lax.dot_general(h_ref[...], UwT_ref[...],
                                 (((1,), (0,)), ((), ())),
                                 preferred_element_type=jnp.float32)
               + Ub_ref[...])

        for jt in range(NT):
            blk = lax.dot_general(r3T[:, E * jt:E * (jt + 1)], WeU_ref[...],
                                  (((1,), (0,)), ((), ())),
                                  preferred_element_type=jnp.float32)
            out_ref[jt * H:(jt + 1) * H, :] = msg[jt * H:(jt + 1) * H, :] + blk


@jax.jit
def kernel(h, edge_attr, adj, W_w, W_b, U_w, U_b):
    edge3 = (edge_attr.reshape(N, NT, H, E)
             .transpose(0, 1, 3, 2)
             .reshape(N, Q, H))

    UwT = U_w.T
    A = W_w[:, :H].T @ UwT
    WeU = W_w[:, H:].T @ UwT
    wbU = (W_b @ UwT)[None, :]
    Ub = U_b[None, :]

    out = pl.pallas_call(
        _body,
        grid=(N // I,),
        in_specs=[
            pl.BlockSpec((I, H), lambda k: (k, 0)),
            pl.BlockSpec((I, N), lambda k: (k, 0)),
            pl.BlockSpec((I, Q, H), lambda k: (k, 0, 0)),
            pl.BlockSpec((N, H), lambda k: (0, 0)),
            pl.BlockSpec((H, H), lambda k: (0, 0)),
            pl.BlockSpec((E, H), lambda k: (0, 0)),
            pl.BlockSpec((1, H), lambda k: (0, 0)),
            pl.BlockSpec((H, H), lambda k: (0, 0)),
            pl.BlockSpec((1, H), lambda k: (0, 0)),
        ],
        out_specs=pl.BlockSpec((N, H), lambda k: (0, 0)),
        out_shape=jax.ShapeDtypeStruct((N, H), jnp.float32),
        scratch_shapes=[
            pltpu.VMEM((N, H), jnp.float32),
            pltpu.VMEM((N, 1), jnp.float32),
            pltpu.VMEM((Q, H), jnp.float32),
        ],
    )(h, adj, edge3, h, A, WeU, wbU, UwT, Ub)
    return out


# speedup vs baseline: 1.0220x; 1.0107x over previous
import jax
import jax.numpy as jnp
from jax import lax
from jax.experimental import pallas as pl
from jax.experimental.pallas import tpu as pltpu

N = 2048
H = 128
E = 4
I = 256
S = 2
NT = N // H
Q = N * E // H
NS = N // S
TS = NT // S
QS = Q // S


def _body(h_blk_ref, adj_ref, edge_ref, h_ref, A_ref, WeU_ref, wbU_ref,
          UwT_ref, Ub_ref, out_ref, aggh_ref, deg_ref, r3_ref):
    k = pl.program_id(0)
    s = pl.program_id(1)

    @pl.when((k == 0) & (s == 0))
    def _init():
        aggh_ref[...] = jnp.zeros_like(aggh_ref)
        deg_ref[...] = jnp.zeros_like(deg_ref)
        r3_ref[...] = jnp.zeros_like(r3_ref)

    mask = (adj_ref[...] == 1).astype(jnp.float32)

    aggh_ref[pl.ds(s * NS, NS), :] += lax.dot_general(
        mask, h_blk_ref[...], (((0,), (0,)), ((), ())),
        preferred_element_type=jnp.float32)

    ones_col = jnp.ones((I, 1), dtype=jnp.float32)
    deg_ref[pl.ds(s * NS, NS), :] += lax.dot_general(
        mask, ones_col, (((0,), (0,)), ((), ())),
        preferred_element_type=jnp.float32)

    maskE = mask.reshape(I, TS, H)
    mask3 = jnp.repeat(maskE, E, axis=1)
    r3_ref[pl.ds(s * QS, QS), :] += jnp.sum(
        mask3 * edge_ref[...], axis=0)

    @pl.when((k == pl.num_programs(0) - 1) & (s == pl.num_programs(1) - 1))
    def _epilogue():
        ii = lax.broadcasted_iota(jnp.int32, (H, H), 0)
        jj = lax.broadcasted_iota(jnp.int32, (H, H), 1)
        ident = (ii == jj).astype(jnp.float32)
        r3T = lax.dot_general(ident, r3_ref[...], (((1,), (1,)), ((), ())),
                              preferred_element_type=jnp.float32)

        msg = (lax.dot_general(aggh_ref[...], A_ref[...],
                               (((1,), (0,)), ((), ())),
                               preferred_element_type=jnp.float32)
               + lax.dot_general(deg_ref[...], wbU_ref[...],
                                 (((1,), (0,)), ((), ())),
                                 preferred_element_type=jnp.float32)
               + lax.dot_general(h_ref[...], UwT_ref[...],
                                 (((1,), (0,)), ((), ())),
                                 preferred_element_type=jnp.float32)
               + Ub_ref[...])

        for jt in range(NT):
            blk = lax.dot_general(r3T[:, E * jt:E * (jt + 1)], WeU_ref[...],
                                  (((1,), (0,)), ((), ())),
                                  preferred_element_type=jnp.float32)
            out_ref[jt * H:(jt + 1) * H, :] = msg[jt * H:(jt + 1) * H, :] + blk


@jax.jit
def kernel(h, edge_attr, adj, W_w, W_b, U_w, U_b):
    edge3 = (edge_attr.reshape(N, NT, H, E)
             .transpose(0, 1, 3, 2)
             .reshape(N, Q, H))

    UwT = U_w.T
    A = W_w[:, :H].T @ UwT
    WeU = W_w[:, H:].T @ UwT
    wbU = (W_b @ UwT)[None, :]
    Ub = U_b[None, :]

    out = pl.pallas_call(
        _body,
        grid=(N // I, S),
        in_specs=[
            pl.BlockSpec((I, H), lambda k, s: (k, 0)),
            pl.BlockSpec((I, NS), lambda k, s: (k, s)),
            pl.BlockSpec((I, QS, H), lambda k, s: (k, s, 0)),
            pl.BlockSpec((N, H), lambda k, s: (0, 0)),
            pl.BlockSpec((H, H), lambda k, s: (0, 0)),
            pl.BlockSpec((E, H), lambda k, s: (0, 0)),
            pl.BlockSpec((1, H), lambda k, s: (0, 0)),
            pl.BlockSpec((H, H), lambda k, s: (0, 0)),
            pl.BlockSpec((1, H), lambda k, s: (0, 0)),
        ],
        out_specs=pl.BlockSpec((N, H), lambda k, s: (0, 0)),
        out_shape=jax.ShapeDtypeStruct((N, H), jnp.float32),
        scratch_shapes=[
            pltpu.VMEM((N, H), jnp.float32),
            pltpu.VMEM((N, 1), jnp.float32),
            pltpu.VMEM((Q, H), jnp.float32),
        ],
    )(h, adj, edge3, h, A, WeU, wbU, UwT, Ub)
    return out


# speedup vs baseline: 1.1037x; 1.0799x over previous
import jax
import jax.numpy as jnp
from jax import lax
from jax.experimental import pallas as pl
from jax.experimental.pallas import tpu as pltpu

N = 2048
H = 128
E = 4
I = 256
NT = N // H
Q = N * E // H
NP = NT // 2


def _body(h_blk_ref, adj_ref, edge_ref, h_ref, Rrep_ref, A_ref, WeU_ref,
          wbU_ref, UwT_ref, Ub_ref, out_ref, aggh_ref, deg_ref, r3_ref):
    k = pl.program_id(0)

    @pl.when(k == 0)
    def _init():
        aggh_ref[...] = jnp.zeros_like(aggh_ref)
        deg_ref[...] = jnp.zeros_like(deg_ref)
        r3_ref[...] = jnp.zeros_like(r3_ref)

    mask = (adj_ref[...] == 1).astype(jnp.float32)
    mask_bf = mask.astype(jnp.bfloat16)

    aggh_ref[...] += lax.dot_general(
        mask, h_blk_ref[...], (((0,), (0,)), ((), ())),
        preferred_element_type=jnp.float32)

    ones_col = jnp.ones((I, 1), dtype=jnp.float32)
    deg_ref[...] += lax.dot_general(
        mask, ones_col, (((0,), (0,)), ((), ())),
        preferred_element_type=jnp.float32)

    rr = lax.broadcasted_iota(jnp.int32, (8 * I, H), 0)
    first_half = (lax.rem(rr, 8) < 4)

    for jp in range(NP):
        m_pair = mask_bf[:, 2 * jp * H:2 * (jp + 1) * H]
        X = lax.dot_general(Rrep_ref[...], m_pair,
                            (((1,), (0,)), ((), ())),
                            preferred_element_type=jnp.float32)
        m3 = jnp.where(first_half, X[:, :H], X[:, H:])
        prod = m3.reshape(I, 8, H) * edge_ref[:, 8 * jp:8 * (jp + 1), :]
        r3_ref[8 * jp:8 * (jp + 1), :] += jnp.sum(prod, axis=0)

    @pl.when(k == pl.num_programs(0) - 1)
    def _epilogue():
        ii = lax.broadcasted_iota(jnp.int32, (H, H), 0)
        jj = lax.broadcasted_iota(jnp.int32, (H, H), 1)
        ident = (ii == jj).astype(jnp.float32)
        r3T = lax.dot_general(ident, r3_ref[...], (((1,), (1,)), ((), ())),
                              preferred_element_type=jnp.float32)

        msg = (lax.dot_general(aggh_ref[...], A_ref[...],
                               (((1,), (0,)), ((), ())),
                               preferred_element_type=jnp.float32)
               + lax.dot_general(deg_ref[...], wbU_ref[...],
                                 (((1,), (0,)), ((), ())),
                                 preferred_element_type=jnp.float32)
               + lax.dot_general(h_ref[...], UwT_ref[...],
                                 (((1,), (0,)), ((), ())),
                                 preferred_element_type=jnp.float32)
               + Ub_ref[...])

        for jt in range(NT):
            blk = lax.dot_general(r3T[:, E * jt:E * (jt + 1)], WeU_ref[...],
                                  (((1,), (0,)), ((), ())),
                                  preferred_element_type=jnp.float32)
            out_ref[jt * H:(jt + 1) * H, :] = msg[jt * H:(jt + 1) * H, :] + blk


@jax.jit
def kernel(h, edge_attr, adj, W_w, W_b, U_w, U_b):
    edge3 = (edge_attr.reshape(N, NT, H, E)
             .transpose(0, 1, 3, 2)
             .reshape(N, Q, H))

    UwT = U_w.T
    A = W_w[:, :H].T @ UwT
    WeU = W_w[:, H:].T @ UwT
    wbU = (W_b @ UwT)[None, :]
    Ub = U_b[None, :]
    Rrep = (jnp.arange(8 * I)[:, None] // 8
            == jnp.arange(I)[None, :]).astype(jnp.bfloat16)

    out = pl.pallas_call(
        _body,
        grid=(N // I,),
        in_specs=[
            pl.BlockSpec((I, H), lambda k: (k, 0)),
            pl.BlockSpec((I, N), lambda k: (k, 0)),
            pl.BlockSpec((I, Q, H), lambda k: (k, 0, 0)),
            pl.BlockSpec((N, H), lambda k: (0, 0)),
            pl.BlockSpec((8 * I, I), lambda k: (0, 0)),
            pl.BlockSpec((H, H), lambda k: (0, 0)),
            pl.BlockSpec((E, H), lambda k: (0, 0)),
            pl.BlockSpec((1, H), lambda k: (0, 0)),
            pl.BlockSpec((H, H), lambda k: (0, 0)),
            pl.BlockSpec((1, H), lambda k: (0, 0)),
        ],
        out_specs=pl.BlockSpec((N, H), lambda k: (0, 0)),
        out_shape=jax.ShapeDtypeStruct((N, H), jnp.float32),
        scratch_shapes=[
            pltpu.VMEM((N, H), jnp.float32),
            pltpu.VMEM((N, 1), jnp.float32),
            pltpu.VMEM((Q, H), jnp.float32),
        ],
    )(h, adj, edge3, h, Rrep, A, WeU, wbU, UwT, Ub)
    return out
